# baseline (speedup 1.0000x reference)
import functools

import jax
import jax.numpy as jnp
from jax import lax
from jax.experimental import pallas as pl
from jax.experimental.pallas import tpu as pltpu
from jax.experimental.pallas import tpu_sc as plsc

EMBED = 128
VW = 16
GB = 128
NBUF = 6

_info = plsc.get_sparse_core_info()
NC, NS = _info.num_cores, _info.num_subcores
NW = NC * NS


def _span(n):
    s = (-(-n // NW) + 7) // 8 * 8
    assert (n - s) % 8 == 0 and s % 8 == 0
    return s


def _build(n_user, n_item, n_cat):
    ns = (n_user, n_item, n_cat)
    spans = tuple(_span(n) for n in ns)
    seg_offs = (0, spans[0], spans[0] + spans[1])
    idx_total = sum(spans)
    mesh = plsc.VectorSubcoreMesh(core_axis_name="c", subcore_axis_name="s")
    out_types = tuple(
        jax.ShapeDtypeStruct((n, EMBED), jnp.float32) for n in ns
    )

    @functools.partial(
        pl.kernel,
        mesh=mesh,
        out_type=out_types,
        scratch_types=[
            pltpu.VMEM((idx_total,), jnp.int32),
            pltpu.VMEM((NBUF, GB, EMBED), jnp.float32),
            pltpu.VMEM((3, EMBED), jnp.float32),
        ]
        + [pltpu.SemaphoreType.DMA] * NBUF,
    )
    def k(xu, xi, xc, table, ou, oi, oc, idx_v, rows_v, table_t, *ssems):
        wid = lax.axis_index("s") * NC + lax.axis_index("c")

        pltpu.sync_copy(table, table_t)
        bases = []
        for x, n, span, soff in zip((xu, xi, xc), ns, spans, seg_offs):
            base = jnp.minimum(wid * span, n - span)
            bases.append(base)
            pltpu.sync_copy(
                x.at[pl.ds(base, span)], idx_v.at[pl.ds(soff, span)]
            )

        trows = [
            [table_t[j, pl.ds(kk * VW, VW)] for kk in range(8)]
            for j in range(3)
        ]

        chunks = []
        for o, base, span, soff in zip((ou, oi, oc), bases, spans, seg_offs):
            n_ch = -(-span // GB)
            for c in range(n_ch):
                off = min(c * GB, span - GB)
                chunks.append((o, base, soff + off, off))

        nch = len(chunks)

        def fill(b, ioff):
            def row(r, carry):
                iv = idx_v[pl.ds(ioff + r, 1)][0]
                for kk in range(8):
                    rows_v[b, r, pl.ds(kk * VW, VW)] = jnp.where(
                        iv == 0,
                        trows[0][kk],
                        jnp.where(iv == 1, trows[1][kk], trows[2][kk]),
                    )
                return carry

            lax.fori_loop(0, GB, row, 0)

        sh = [None] * NBUF
        for ci in range(nch):
            b = ci % NBUF
            o, base, ioff, off = chunks[ci]
            if ci >= NBUF:
                sh[b].wait()
            fill(b, ioff)
            sh[b] = pltpu.async_copy(
                rows_v.at[b], o.at[pl.ds(base + off, GB)], ssems[b]
            )
        for ci in range(max(0, nch - NBUF), nch):
            sh[ci % NBUF].wait()

    return k


_embed3 = _build(100000, 50000, 50000)


def kernel(x_user, x_item, x_category, table):
    ou, oi, oc = _embed3(
        x_user.astype(jnp.int32),
        x_item.astype(jnp.int32),
        x_category.astype(jnp.int32),
        table,
    )
    return (ou, ou, oi, oi, oc, oc)


# speedup vs baseline: 1.5733x; 1.5733x over previous
import functools

import jax
import jax.numpy as jnp
from jax import lax
from jax.experimental import pallas as pl
from jax.experimental.pallas import tpu as pltpu
from jax.experimental.pallas import tpu_sc as plsc

EMBED = 128
GB = 128
NBUF = 6

_info = plsc.get_sparse_core_info()
NC, NS = _info.num_cores, _info.num_subcores
NW = NC * NS


def _span(n):
    s = (-(-n // NW) + 7) // 8 * 8
    assert (n - s) % 8 == 0 and s % 8 == 0
    return s


def _build(n_user, n_item, n_cat):
    ns = (n_user, n_item, n_cat)
    spans = tuple(_span(n) for n in ns)
    seg_offs = (0, spans[0], spans[0] + spans[1])
    idx_total = sum(spans)
    mesh = plsc.VectorSubcoreMesh(core_axis_name="c", subcore_axis_name="s")
    out_types = tuple(
        jax.ShapeDtypeStruct((n, EMBED), jnp.float32) for n in ns
    )

    @functools.partial(
        pl.kernel,
        mesh=mesh,
        out_type=out_types,
        scratch_types=[
            pltpu.VMEM((idx_total,), jnp.int32),
            pltpu.VMEM((NBUF, GB, EMBED), jnp.float32),
            pltpu.VMEM_SHARED((3, EMBED), jnp.float32),
        ]
        + [pltpu.SemaphoreType.DMA] * NBUF
        + [pltpu.SemaphoreType.DMA] * NBUF,
    )
    def k(xu, xi, xc, table, ou, oi, oc, idx_v, rows_v, table_s, *sems):
        gsems, ssems = sems[:NBUF], sems[NBUF:]
        wid = lax.axis_index("s") * NC + lax.axis_index("c")

        @pl.when(lax.axis_index("s") == 0)
        def _():
            pltpu.sync_copy(table, table_s)

        bases = []
        for x, n, span, soff in zip((xu, xi, xc), ns, spans, seg_offs):
            base = jnp.minimum(wid * span, n - span)
            bases.append(base)
            pltpu.sync_copy(
                x.at[pl.ds(base, span)], idx_v.at[pl.ds(soff, span)]
            )

        plsc.subcore_barrier()

        chunks = []
        for o, base, span, soff in zip((ou, oi, oc), bases, spans, seg_offs):
            n_ch = -(-span // GB)
            for c in range(n_ch):
                off = min(c * GB, span - GB)
                chunks.append((o, base, soff + off, off))

        nch = len(chunks)

        def fire_gather(ci):
            _, _, ioff, _ = chunks[ci]
            return pltpu.async_copy(
                table_s.at[idx_v.at[pl.ds(ioff, GB)]],
                rows_v.at[ci % NBUF],
                gsems[ci % NBUF],
            )

        gh = [None] * NBUF
        sh = [None] * NBUF
        for ci in range(min(NBUF, nch)):
            gh[ci] = fire_gather(ci)
        for ci in range(nch):
            b = ci % NBUF
            o, base, _, off = chunks[ci]
            gh[b].wait()
            sh[b] = pltpu.async_copy(
                rows_v.at[b], o.at[pl.ds(base + off, GB)], ssems[b]
            )
            if ci + NBUF < nch:
                sh[b].wait()
                gh[b] = fire_gather(ci + NBUF)
        for ci in range(max(0, nch - NBUF), nch):
            sh[ci % NBUF].wait()

    return k


_embed3 = _build(100000, 50000, 50000)


def kernel(x_user, x_item, x_category, table):
    ou, oi, oc = _embed3(
        x_user.astype(jnp.int32),
        x_item.astype(jnp.int32),
        x_category.astype(jnp.int32),
        table,
    )
    return (ou, ou, oi, oi, oc, oc)
